# baseline (speedup 1.0000x reference)
import functools
import math

import jax
import jax.numpy as jnp
from jax import lax
from jax.experimental import pallas as pl
from jax.experimental.pallas import tpu as pltpu
from jax.experimental.pallas import tpu_sc as plsc

D = 1024
B = 32768
SCALE = math.sqrt(D)

NC = 2
NS = 16
NW = NC * NS
B_PER_W = B // NW
CHUNK = 16
NBUF = 4
NSTEP = B_PER_W // CHUNK
LANES = 16

_mesh = plsc.VectorSubcoreMesh(core_axis_name="c", subcore_axis_name="s")


@functools.partial(
    pl.kernel,
    mesh=_mesh,
    out_type=jax.ShapeDtypeStruct((B, D), jnp.float32),
    scratch_types=[
        pltpu.VMEM((B_PER_W,), jnp.int32),
        pltpu.VMEM((NBUF * CHUNK, D), jnp.float32),
    ]
    + [pltpu.SemaphoreType.DMA] * (NBUF + 2),
)
def _gather_scale(table_hbm, idx_hbm, out_hbm, idx_v, rows_v, *sems):
    gsem = sems[:NBUF]
    ssem = sems[NBUF:]
    wid = lax.axis_index("s") * NC + lax.axis_index("c")
    base = wid * B_PER_W
    pltpu.sync_copy(idx_hbm.at[pl.ds(base, B_PER_W)], idx_v)

    def gather(buf, g):
        return pltpu.make_async_copy(
            table_hbm.at[idx_v.at[pl.ds(g * CHUNK, CHUNK)]],
            rows_v.at[pl.ds(buf * CHUNK, CHUNK)], gsem[buf])

    def store_pair(pair, g):
        return pltpu.make_async_copy(
            rows_v.at[pl.ds(pair * 2 * CHUNK, 2 * CHUNK)],
            out_hbm.at[pl.ds(base + g * CHUNK, 2 * CHUNK)], ssem[pair])

    def scale(buf):
        @pl.loop(0, CHUNK, unroll=2)
        def _row(r):
            for c in range(0, D, LANES):
                rows_v[buf * CHUNK + r, pl.ds(c, LANES)] = (
                    rows_v[buf * CHUNK + r, pl.ds(c, LANES)] * SCALE)

    gather(0, 0).start()
    gather(1, 1).start()

    @pl.loop(0, NSTEP, step=NBUF)
    def _step(j):
        gather(0, j).wait()

        @pl.when(j >= 2)
        def _drain_b():
            store_pair(1, j - 2).wait()

        gather(2, j + 2).start()
        scale(0)

        gather(1, j + 1).wait()
        gather(3, j + 3).start()
        scale(1)
        store_pair(0, j).start()

        gather(2, j + 2).wait()

        @pl.when(j + 4 < NSTEP)
        def _prefetch_a():
            store_pair(0, j).wait()
            gather(0, j + 4).start()

        scale(2)

        gather(3, j + 3).wait()

        @pl.when(j + 5 < NSTEP)
        def _prefetch_a2():
            gather(1, j + 5).start()

        scale(3)
        store_pair(1, j + 2).start()

    store_pair(0, NSTEP - 4).wait()
    store_pair(1, NSTEP - 2).wait()


def kernel(x, table):
    idx = x.reshape(B).astype(jnp.int32)
    out = _gather_scale(table, idx)
    return out.reshape(x.shape[0], x.shape[1], D)


# speedup vs baseline: 1.0659x; 1.0659x over previous
import functools
import math

import jax
import jax.numpy as jnp
from jax import lax
from jax.experimental import pallas as pl
from jax.experimental.pallas import tpu as pltpu
from jax.experimental.pallas import tpu_sc as plsc

D = 1024
B = 32768
SCALE = math.sqrt(D)

NC = 2
NS = 16
NW = NC * NS
B_PER_W = B // NW
CHUNK = 16
NBUF = 4
NSTEP = B_PER_W // CHUNK
LANES = 16

_mesh = plsc.VectorSubcoreMesh(core_axis_name="c", subcore_axis_name="s")


@functools.partial(
    pl.kernel,
    mesh=_mesh,
    out_type=jax.ShapeDtypeStruct((B, D), jnp.float32),
    scratch_types=[
        pltpu.VMEM((B_PER_W,), jnp.int32),
        pltpu.VMEM((NBUF, CHUNK, D), jnp.float32),
    ]
    + [pltpu.SemaphoreType.DMA] * (2 * NBUF),
)
def _gather_scale(table_hbm, idx_hbm, out_hbm, idx_v, rows_v, *sems):
    gsem = sems[:NBUF]
    ssem = sems[NBUF:]
    wid = lax.axis_index("s") * NC + lax.axis_index("c")
    base = wid * B_PER_W
    pltpu.sync_copy(idx_hbm.at[pl.ds(base, B_PER_W)], idx_v)

    def gather(buf, g):
        return pltpu.make_async_copy(
            table_hbm.at[idx_v.at[pl.ds(g * CHUNK, CHUNK)]],
            rows_v.at[buf], gsem[buf])

    def store(buf, g):
        return pltpu.make_async_copy(
            rows_v.at[buf], out_hbm.at[pl.ds(base + g * CHUNK, CHUNK)],
            ssem[buf])

    def scale(buf):
        @pl.loop(0, CHUNK, unroll=4)
        def _row(r):
            for c in range(0, D, LANES):
                rows_v[buf, r, pl.ds(c, LANES)] = (
                    rows_v[buf, r, pl.ds(c, LANES)] * SCALE)

    gather(0, 0).start()
    gather(1, 1).start()

    @pl.loop(0, NSTEP, step=NBUF)
    def _step(j):
        for s in range(NBUF):
            buf = s
            g = j + s
            gather(buf, g).wait()

            pbuf = (s + 2) % NBUF

            @pl.when(g + 2 < NSTEP)
            def _prefetch():
                @pl.when(g >= 2)
                def _drain():
                    store(pbuf, g - 2).wait()

                gather(pbuf, g + 2).start()

            scale(buf)
            store(buf, g).start()

    for s in range(NBUF):
        store(s, NSTEP - NBUF + s).wait()


def kernel(x, table):
    idx = x.reshape(B).astype(jnp.int32)
    out = _gather_scale(table, idx)
    return out.reshape(x.shape[0], x.shape[1], D)


# speedup vs baseline: 1.2289x; 1.1529x over previous
import functools
import math

import jax
import jax.numpy as jnp
from jax import lax
from jax.experimental import pallas as pl
from jax.experimental.pallas import tpu as pltpu
from jax.experimental.pallas import tpu_sc as plsc

D = 1024
B = 32768
SCALE = math.sqrt(D)

NC = 2
NS = 16
NW = NC * NS
B_PER_W = B // NW
CHUNK = 16
NBUF = 4
NSTEP = B_PER_W // CHUNK
LANES = 16

_mesh = plsc.VectorSubcoreMesh(core_axis_name="c", subcore_axis_name="s")


@functools.partial(
    pl.kernel,
    mesh=_mesh,
    out_type=jax.ShapeDtypeStruct((B, D), jnp.float32),
    scratch_types=[
        pltpu.VMEM((B_PER_W,), jnp.int32),
        pltpu.VMEM((NBUF, CHUNK, D), jnp.float32),
    ]
    + [pltpu.SemaphoreType.DMA] * (2 * NBUF),
)
def _gather_scale(table_hbm, idx_hbm, out_hbm, idx_v, rows_v, *sems):
    gsem = sems[:NBUF]
    ssem = sems[NBUF:]
    wid = lax.axis_index("s") * NC + lax.axis_index("c")
    base = wid * B_PER_W
    pltpu.sync_copy(idx_hbm.at[pl.ds(base, B_PER_W)], idx_v)

    def gather(buf, g):
        return pltpu.make_async_copy(
            table_hbm.at[idx_v.at[pl.ds(g * CHUNK, CHUNK)]],
            rows_v.at[buf], gsem[buf])

    def store(buf, g):
        return pltpu.make_async_copy(
            rows_v.at[buf], out_hbm.at[pl.ds(base + g * CHUNK, CHUNK)],
            ssem[buf])

    def scale(buf):
        @pl.loop(0, CHUNK, unroll=2)
        def _row(r):
            for c in range(0, D, LANES):
                rows_v[buf, r, pl.ds(c, LANES)] = (
                    rows_v[buf, r, pl.ds(c, LANES)] * SCALE)

    gather(0, 0).start()
    gather(1, 1).start()

    @pl.loop(0, NSTEP, step=NBUF)
    def _step(j):
        for s in range(NBUF):
            buf = s
            g = j + s
            gather(buf, g).wait()

            pbuf = (s + 2) % NBUF

            @pl.when(g + 2 < NSTEP)
            def _prefetch():
                @pl.when(g >= 2)
                def _drain():
                    store(pbuf, g - 2).wait()

                gather(pbuf, g + 2).start()

            scale(buf)
            store(buf, g).start()

    for s in range(NBUF):
        store(s, NSTEP - NBUF + s).wait()


def kernel(x, table):
    idx = x.reshape(B).astype(jnp.int32)
    out = _gather_scale(table, idx)
    return out.reshape(x.shape[0], x.shape[1], D)


# speedup vs baseline: 1.2345x; 1.0046x over previous
import functools
import math

import jax
import jax.numpy as jnp
from jax import lax
from jax.experimental import pallas as pl
from jax.experimental.pallas import tpu as pltpu
from jax.experimental.pallas import tpu_sc as plsc

D = 1024
B = 32768
SCALE = math.sqrt(D)

NC = 2
NS = 16
NW = NC * NS
B_PER_W = B // NW
CHUNK = 16
NBUF = 4
NSTEP = B_PER_W // CHUNK
LANES = 16

_mesh = plsc.VectorSubcoreMesh(core_axis_name="c", subcore_axis_name="s")


@functools.partial(
    pl.kernel,
    mesh=_mesh,
    out_type=jax.ShapeDtypeStruct((B, D), jnp.float32),
    scratch_types=[
        pltpu.VMEM((B_PER_W,), jnp.int32),
        pltpu.VMEM((NBUF, CHUNK, D), jnp.float32),
    ]
    + [pltpu.SemaphoreType.DMA] * (2 * NBUF),
)
def _gather_scale(table_hbm, idx_hbm, out_hbm, idx_v, rows_v, *sems):
    gsem = sems[:NBUF]
    ssem = sems[NBUF:]
    wid = lax.axis_index("s") * NC + lax.axis_index("c")
    base = wid * B_PER_W
    pltpu.sync_copy(idx_hbm.at[pl.ds(base, B_PER_W)], idx_v)

    def gather(buf, g):
        return pltpu.make_async_copy(
            table_hbm.at[idx_v.at[pl.ds(g * CHUNK, CHUNK)]],
            rows_v.at[buf], gsem[buf])

    def store(buf, g):
        return pltpu.make_async_copy(
            rows_v.at[buf], out_hbm.at[pl.ds(base + g * CHUNK, CHUNK)],
            ssem[buf])

    def scale(buf):
        @pl.loop(0, CHUNK)
        def _row(r):
            for c in range(0, D, LANES):
                rows_v[buf, r, pl.ds(c, LANES)] = (
                    rows_v[buf, r, pl.ds(c, LANES)] * SCALE)

    gather(0, 0).start()
    gather(1, 1).start()

    @pl.loop(0, NSTEP, step=NBUF)
    def _step(j):
        for s in range(NBUF):
            buf = s
            g = j + s
            gather(buf, g).wait()

            pbuf = (s + 2) % NBUF

            @pl.when(g + 2 < NSTEP)
            def _prefetch():
                @pl.when(g >= 2)
                def _drain():
                    store(pbuf, g - 2).wait()

                gather(pbuf, g + 2).start()

            scale(buf)
            store(buf, g).start()

    for s in range(NBUF):
        store(s, NSTEP - NBUF + s).wait()


def kernel(x, table):
    idx = x.reshape(B).astype(jnp.int32)
    out = _gather_scale(table, idx)
    return out.reshape(x.shape[0], x.shape[1], D)
